# baseline (speedup 1.0000x reference)
import functools

import jax
import jax.numpy as jnp
from jax import lax
from jax.experimental import pallas as pl
from jax.experimental.pallas import tpu as pltpu
from jax.experimental.pallas import tpu_sc as plsc

N_NODES_ = 10000
N_EDGES_ = 320000
D_ = 128

NC = 2
NS = 16
NW = NC * NS
E_PER_TILE = N_EDGES_ // NW
CHUNK = 128
N_CHUNKS = -(-E_PER_TILE // CHUNK)
E_PAD = N_CHUNKS * CHUNK
TRASH_ROW = N_NODES_
ACC_ROWS = N_NODES_ + 8
ROWS_PER_TILE = 624
TAIL_BASE = NS * ROWS_PER_TILE


def _sc_body(idx_hbm, emb_hbm, zeros_hbm, out_hbm,
             blk0, blk1, rows0, rows1, acc, semi0, semi1, semr0, semr1):
  c = lax.axis_index("c")
  s = lax.axis_index("s")
  t = c * NS + s
  row_base = s * ROWS_PER_TILE

  pltpu.sync_copy(zeros_hbm.at[pl.ds(row_base, ROWS_PER_TILE)],
                  acc.at[pl.ds(row_base, ROWS_PER_TILE)])

  @pl.when(s == 0)
  def _init_tail():
    pltpu.sync_copy(zeros_hbm.at[pl.ds(TAIL_BASE, ACC_ROWS - TAIL_BASE)],
                    acc.at[pl.ds(TAIL_BASE, ACC_ROWS - TAIL_BASE)])

  plsc.subcore_barrier()

  blks = (blk0, blk1)
  rows = (rows0, rows1)
  semis = (semi0, semi1)
  semrs = (semr0, semr1)

  pltpu.async_copy(idx_hbm.at[t, 0], blk0, semi0)
  pltpu.make_async_copy(idx_hbm.at[t, 0], blk0, semi0).wait()
  pltpu.async_copy(emb_hbm.at[blk0.at[0]], rows0, semr0)
  pltpu.async_copy(idx_hbm.at[t, 1], blk1, semi1)

  def step(j, b, issue_gather, issue_idx):
    nb = 1 - b
    pltpu.make_async_copy(emb_hbm.at[blks[b].at[0]], rows[b], semrs[b]).wait()
    if issue_gather:
      pltpu.make_async_copy(idx_hbm.at[t, j + 1], blks[nb], semis[nb]).wait()
      pltpu.async_copy(emb_hbm.at[blks[nb].at[0]], rows[nb], semrs[nb])
    pltpu.sync_copy(rows[b], acc.at[blks[b].at[1]], add=True)
    if issue_idx:
      pltpu.async_copy(idx_hbm.at[t, j + 2], blks[b], semis[b])

  def chunk_pair(i, carry):
    step(2 * i, 0, True, True)
    step(2 * i + 1, 1, True, True)
    return carry

  K = (N_CHUNKS - 3) // 2
  lax.fori_loop(0, K, chunk_pair, 0)
  for j in range(2 * K, N_CHUNKS):
    step(j, j % 2, j + 1 < N_CHUNKS, j + 2 < N_CHUNKS)

  plsc.subcore_barrier()
  pltpu.sync_copy(acc.at[pl.ds(row_base, ROWS_PER_TILE)],
                  out_hbm.at[c, pl.ds(row_base, ROWS_PER_TILE)])

  @pl.when(s == 0)
  def _write_tail():
    pltpu.sync_copy(acc.at[pl.ds(TAIL_BASE, N_NODES_ - TAIL_BASE)],
                    out_hbm.at[c, pl.ds(TAIL_BASE, N_NODES_ - TAIL_BASE)])


@functools.partial(
    pl.kernel,
    out_type=jax.ShapeDtypeStruct((NC, N_NODES_, D_), jnp.float32),
    mesh=plsc.VectorSubcoreMesh(core_axis_name="c", subcore_axis_name="s"),
    scratch_types=[
        pltpu.VMEM((2, CHUNK), jnp.int32),
        pltpu.VMEM((2, CHUNK), jnp.int32),
        pltpu.VMEM((CHUNK, D_), jnp.float32),
        pltpu.VMEM((CHUNK, D_), jnp.float32),
        pltpu.VMEM_SHARED((ACC_ROWS, D_), jnp.float32),
        pltpu.SemaphoreType.DMA,
        pltpu.SemaphoreType.DMA,
        pltpu.SemaphoreType.DMA,
        pltpu.SemaphoreType.DMA,
    ],
)
def _sc_aggregate(idx_hbm, emb_hbm, zeros_hbm, out_hbm,
                  blk0, blk1, rows0, rows1, acc, semi0, semi1, semr0, semr1):
  _sc_body(idx_hbm, emb_hbm, zeros_hbm, out_hbm,
           blk0, blk1, rows0, rows1, acc, semi0, semi1, semr0, semr1)


def _add_body(a_ref, b_ref, o_ref):
  o_ref[...] = a_ref[...] + b_ref[...]


def _combine(p0, p1):
  blk = 1000
  return pl.pallas_call(
      _add_body,
      out_shape=jax.ShapeDtypeStruct((N_NODES_, D_), jnp.float32),
      grid=(N_NODES_ // blk,),
      in_specs=[pl.BlockSpec((blk, D_), lambda i: (i, 0)),
                pl.BlockSpec((blk, D_), lambda i: (i, 0))],
      out_specs=pl.BlockSpec((blk, D_), lambda i: (i, 0)),
  )(p0, p1)


def kernel(mode, edge_index, entity_embed):
  del mode
  src = edge_index[0].reshape(NW, E_PER_TILE)
  dst = edge_index[1].reshape(NW, E_PER_TILE)
  pad = E_PAD - E_PER_TILE
  src_p = jnp.pad(src, ((0, 0), (0, pad))).reshape(NW, N_CHUNKS, CHUNK)
  dst_p = jnp.pad(dst, ((0, 0), (0, pad)),
                  constant_values=TRASH_ROW).reshape(NW, N_CHUNKS, CHUNK)
  idx_packed = jnp.stack([src_p, dst_p], axis=2)
  zeros = jnp.zeros((ACC_ROWS, D_), jnp.float32)
  partials = _sc_aggregate(idx_packed, entity_embed, zeros)
  return _combine(partials[0], partials[1])
